# baseline (speedup 1.0000x reference)
import functools

import jax
import jax.numpy as jnp
from jax import lax
from jax.experimental import pallas as pl
from jax.experimental.pallas import tpu as pltpu
from jax.experimental.pallas import tpu_sc as plsc

_B = 4096
_HW = 16384
_NT = 16
_CHUNK = _B // _NT
_L = 16
_HALF = _CHUNK // 2
_ROWS = 2 * 32 * _HW // _L
_INV_N = 1.0 / _B


@functools.partial(
    pl.kernel,
    mesh=plsc.VectorSubcoreMesh(core_axis_name="c", subcore_axis_name="s",
                                num_cores=1),
    out_type=jax.ShapeDtypeStruct((1,), jnp.float32),
    compiler_params=pltpu.CompilerParams(needs_layout_passes=False,
                                         use_tc_tiling_on_sc=False),
    scratch_types=[
        pltpu.VMEM((_CHUNK,), jnp.int32),
        pltpu.VMEM((2, _HALF), jnp.int32),
        pltpu.VMEM((2, _HALF), jnp.int32),
        pltpu.VMEM((_CHUNK,), jnp.int32),
        pltpu.VMEM((2, _HALF, _L), jnp.float32),
        pltpu.VMEM((2, _HALF, _L), jnp.float32),
        pltpu.VMEM((2, 2, _HALF), jnp.float32),
        pltpu.VMEM((_L,), jnp.float32),
        pltpu.VMEM((_NT, _L), jnp.float32),
        pltpu.VMEM((_L,), jnp.float32),
        pltpu.VMEM_SHARED((_NT, _L), jnp.float32),
        pltpu.SemaphoreType.DMA,
        pltpu.SemaphoreType.DMA,
        pltpu.SemaphoreType.DMA,
    ],
)
def _sc_gather_l1(table_hbm, ind_hbm, tgt_hbm, out_hbm,
                  ind_v, row0_v, row1_v, lane_v, g0_v, g1_v, tb_v,
                  part_v, all_v, res_v, shared, sem0, sem1, sem2):
    wid = lax.axis_index("s")
    base = wid * _CHUNK

    pltpu.sync_copy(ind_hbm.at[pl.ds(base, _CHUNK)], ind_v)
    t_cps = [pltpu.async_copy(tgt_hbm.at[wid], tb_v, sem1)]

    def _rows(j, carry):
        sl = pl.ds(j * _L, _L)
        iv = ind_v[sl]
        f0 = iv + iv - jnp.bitwise_and(iv, jnp.int32(_HW - 1))
        r0 = lax.shift_right_logical(f0, 4)
        h = j // (_HALF // _L)
        hsl = pl.ds((j % (_HALF // _L)) * _L, _L)
        row0_v[h, hsl] = r0
        row1_v[h, hsl] = r0 + jnp.int32(_HW // _L)
        lane_v[sl] = jnp.bitwise_and(f0, jnp.int32(_L - 1))
        return carry

    iota = lax.iota(jnp.int32, _L)

    def _l1(j, acc):
        h = j // (_HALF // _L)
        jj = j % (_HALF // _L)
        k = iota + jj * _L
        lane = lane_v[pl.ds(j * _L, _L)]
        hsl = pl.ds(jj * _L, _L)
        v0 = plsc.load_gather(g0_v.at[h], [k, lane])
        v1 = plsc.load_gather(g1_v.at[h], [k, lane])
        return acc + jnp.abs(v0 - tb_v[h, 0, hsl]) + jnp.abs(v1 - tb_v[h, 1, hsl])

    nj = _HALF // _L
    g_sems = (sem0, sem2)
    g_cps = []
    for h in range(2):
        lax.fori_loop(h * nj, (h + 1) * nj, _rows, 0, unroll=False)
        g_cps.append(pltpu.async_copy(table_hbm.at[row0_v.at[h]],
                                      g0_v.at[h], g_sems[h]))
        g_cps.append(pltpu.async_copy(table_hbm.at[row1_v.at[h]],
                                      g1_v.at[h], g_sems[h]))
    for cp in t_cps:
        cp.wait()

    acc = jnp.zeros((_L,), jnp.float32)
    for h in range(2):
        g_cps[2 * h].wait()
        g_cps[2 * h + 1].wait()
        acc = lax.fori_loop(h * nj, (h + 1) * nj, _l1, acc, unroll=2)

    part_v[...] = acc
    pltpu.sync_copy(part_v, shared.at[wid])
    plsc.subcore_barrier()

    @pl.when(wid == 0)
    def _final():
        pltpu.sync_copy(shared, all_v)

        def _sum(i, tot):
            return tot + all_v[i, :]

        tot = lax.fori_loop(0, _NT, _sum, jnp.zeros((_L,), jnp.float32),
                            unroll=False)
        s = jnp.sum(tot) * jnp.float32(_INV_N)
        res_v[...] = jnp.full((_L,), s, jnp.float32)
        pltpu.sync_copy(res_v.at[pl.ds(0, 1)], out_hbm)


def kernel(output, mask, ind, target):
    del mask
    table = output.reshape(_ROWS, _L)
    ind32 = ind.astype(jnp.int32)
    tview = jnp.transpose(target.reshape(32, 128, 2), (0, 2, 1)).reshape(16, 2, 2, 128)
    loss = _sc_gather_l1(table, ind32, tview)
    return loss.reshape(())


# speedup vs baseline: 1.0045x; 1.0045x over previous
import functools

import jax
import jax.numpy as jnp
from jax import lax
from jax.experimental import pallas as pl
from jax.experimental.pallas import tpu as pltpu
from jax.experimental.pallas import tpu_sc as plsc

_B = 4096
_HW = 16384
_NT = 16
_CHUNK = _B // _NT
_L = 16
_HALF = _CHUNK // 2
_ROWS = 2 * 32 * _HW // _L
_INV_N = 1.0 / _B


@functools.partial(
    pl.kernel,
    mesh=plsc.VectorSubcoreMesh(core_axis_name="c", subcore_axis_name="s",
                                num_cores=1),
    out_type=jax.ShapeDtypeStruct((1,), jnp.float32),
    compiler_params=pltpu.CompilerParams(needs_layout_passes=False,
                                         use_tc_tiling_on_sc=False),
    scratch_types=[
        pltpu.VMEM((_CHUNK,), jnp.int32),
        pltpu.VMEM((2, _HALF), jnp.int32),
        pltpu.VMEM((2, _HALF), jnp.int32),
        pltpu.VMEM((_CHUNK,), jnp.int32),
        pltpu.VMEM((2, _HALF, _L), jnp.float32),
        pltpu.VMEM((2, _HALF, _L), jnp.float32),
        pltpu.VMEM((2, 2, _HALF), jnp.float32),
        pltpu.VMEM((_L,), jnp.float32),
        pltpu.VMEM((_NT, _L), jnp.float32),
        pltpu.VMEM((_L,), jnp.float32),
        pltpu.VMEM_SHARED((_NT, _L), jnp.float32),
        pltpu.SemaphoreType.DMA,
        pltpu.SemaphoreType.DMA,
        pltpu.SemaphoreType.DMA,
        pltpu.SemaphoreType.DMA,
    ],
)
def _sc_gather_l1(table_hbm, ind_hbm, tgt_hbm, out_hbm,
                  ind_v, row0_v, row1_v, lane_v, g0_v, g1_v, tb_v,
                  part_v, all_v, res_v, shared, sem0, sem1, sem2, sem3):
    wid = lax.axis_index("s")
    base = wid * _CHUNK

    i_cps = [pltpu.async_copy(ind_hbm.at[pl.ds(base + h * _HALF, _HALF)],
                              ind_v.at[pl.ds(h * _HALF, _HALF)], sem3)
             for h in range(2)]
    t_cps = [pltpu.async_copy(tgt_hbm.at[wid], tb_v, sem1)]

    def _rows(j, carry):
        sl = pl.ds(j * _L, _L)
        iv = ind_v[sl]
        f0 = iv + iv - jnp.bitwise_and(iv, jnp.int32(_HW - 1))
        r0 = lax.shift_right_logical(f0, 4)
        h = j // (_HALF // _L)
        hsl = pl.ds((j % (_HALF // _L)) * _L, _L)
        row0_v[h, hsl] = r0
        row1_v[h, hsl] = r0 + jnp.int32(_HW // _L)
        lane_v[sl] = jnp.bitwise_and(f0, jnp.int32(_L - 1))
        return carry

    iota = lax.iota(jnp.int32, _L)

    def _l1(j, acc):
        h = j // (_HALF // _L)
        jj = j % (_HALF // _L)
        k = iota + jj * _L
        lane = lane_v[pl.ds(j * _L, _L)]
        hsl = pl.ds(jj * _L, _L)
        v0 = plsc.load_gather(g0_v.at[h], [k, lane])
        v1 = plsc.load_gather(g1_v.at[h], [k, lane])
        return acc + jnp.abs(v0 - tb_v[h, 0, hsl]) + jnp.abs(v1 - tb_v[h, 1, hsl])

    nj = _HALF // _L
    g_sems = (sem0, sem2)
    g_cps = []
    for h in range(2):
        i_cps[h].wait()
        lax.fori_loop(h * nj, (h + 1) * nj, _rows, 0, unroll=False)
        g_cps.append(pltpu.async_copy(table_hbm.at[row0_v.at[h]],
                                      g0_v.at[h], g_sems[h]))
        g_cps.append(pltpu.async_copy(table_hbm.at[row1_v.at[h]],
                                      g1_v.at[h], g_sems[h]))
    for cp in t_cps:
        cp.wait()

    acc = jnp.zeros((_L,), jnp.float32)
    for h in range(2):
        g_cps[2 * h].wait()
        g_cps[2 * h + 1].wait()
        acc = lax.fori_loop(h * nj, (h + 1) * nj, _l1, acc, unroll=False)

    part_v[...] = acc
    pltpu.sync_copy(part_v, shared.at[wid])
    plsc.subcore_barrier()

    @pl.when(wid == 0)
    def _final():
        pltpu.sync_copy(shared, all_v)

        def _sum(i, tot):
            return tot + all_v[i, :]

        tot = lax.fori_loop(0, _NT, _sum, jnp.zeros((_L,), jnp.float32),
                            unroll=False)
        s = jnp.sum(tot) * jnp.float32(_INV_N)
        res_v[...] = jnp.full((_L,), s, jnp.float32)
        pltpu.sync_copy(res_v.at[pl.ds(0, 1)], out_hbm)


def kernel(output, mask, ind, target):
    del mask
    table = output.reshape(_ROWS, _L)
    ind32 = ind.astype(jnp.int32)
    tview = jnp.transpose(target.reshape(32, 128, 2), (0, 2, 1)).reshape(16, 2, 2, 128)
    loss = _sc_gather_l1(table, ind32, tview)
    return loss.reshape(())
